# Initial kernel scaffold; baseline (speedup 1.0000x reference)
import jax
import jax.numpy as jnp
from jax.experimental import pallas as pl


def kernel(edge_index, node_embeddings, gc_weight, curvature):
    raise NotImplementedError("write your pallas kernel here")


# speedup vs baseline: 5.6495x; 5.6495x over previous
import functools

import jax
import jax.numpy as jnp
from jax import lax
from jax.experimental import pallas as pl
from jax.experimental.pallas import tpu as pltpu
from jax.experimental.pallas import tpu_sc as plsc

N = 10000
E = 320000
D = 128
DA = 144
NC = 2
NS = 16
NW = NC * NS
EPT = E // NW
K = 80
NCHUNK = EPT // K
RPT = N // NS


def _scale_kernel(emb_ref, c_ref, out_ref):
    y = emb_ref[...]
    c = c_ref[0, 0]
    rc = jnp.sqrt(c)
    n = jnp.sqrt(jnp.sum(y * y, axis=1, keepdims=True))
    n = jnp.clip(n, 1e-10)
    z = rc * n
    atanh = 0.5 * jnp.log((1.0 + z) / (1.0 - z))
    t = (2.0 / rc) * atanh / n * y
    rows = y.shape[0]
    ones = jnp.ones((rows, 1), dtype=y.dtype)
    pad = jnp.zeros((rows, DA - D - 1), dtype=y.dtype)
    out_ref[...] = jnp.concatenate([t, ones, pad], axis=1)


def _seg_sum_kernel(taug, src_h, dst_h, zeros_h, out, src_v, dst_v, rows_v,
                    acc_sh, sem):
    cid = lax.axis_index("c")
    sid = lax.axis_index("s")
    tid = cid * NS + sid

    pltpu.sync_copy(zeros_h, acc_sh.at[pl.ds(sid * RPT, RPT)])
    plsc.subcore_barrier()

    ebase = tid * EPT

    def body(i, carry):
        base = pl.multiple_of(ebase + i * K, 8)
        pltpu.sync_copy(src_h.at[pl.ds(base, K)], src_v)
        pltpu.sync_copy(dst_h.at[pl.ds(base, K)], dst_v)
        pltpu.async_copy(taug.at[src_v], rows_v, sem).wait()
        pltpu.sync_copy(rows_v, acc_sh.at[dst_v], add=True)
        return carry

    lax.fori_loop(0, NCHUNK, body, 0)

    plsc.subcore_barrier()
    pltpu.sync_copy(acc_sh.at[pl.ds(sid * RPT, RPT)],
                    out.at[cid, pl.ds(sid * RPT, RPT)])


def _finish_kernel(part_ref, w_ref, c_ref, out_ref):
    p = part_ref[...]
    s = p[0] + p[1]
    c = c_ref[0, 0]
    rc = jnp.sqrt(c)
    agg = s[:, :D]
    cnt = jnp.clip(s[:, D:D + 1], 1.0)
    neigh = jnp.dot(agg, w_ref[...], preferred_element_type=jnp.float32) / cnt
    m = jnp.sqrt(jnp.sum(neigh * neigh, axis=1, keepdims=True))
    m = jnp.clip(m, 1e-10)
    out_ref[...] = jnp.tanh(rc * m * 0.5) * neigh / (rc * m)


def kernel(edge_index, node_embeddings, gc_weight, curvature):
    c2d = curvature.reshape(1, 1).astype(jnp.float32)
    src = edge_index[0].astype(jnp.int32)
    dst = edge_index[1].astype(jnp.int32)

    rows_blk = 1000
    taug = pl.pallas_call(
        _scale_kernel,
        grid=(N // rows_blk,),
        in_specs=[
            pl.BlockSpec((rows_blk, D), lambda i: (i, 0)),
            pl.BlockSpec(memory_space=pltpu.SMEM),
        ],
        out_specs=pl.BlockSpec((rows_blk, DA), lambda i: (i, 0)),
        out_shape=jax.ShapeDtypeStruct((N, DA), jnp.float32),
    )(node_embeddings, c2d)

    zeros_h = jnp.zeros((RPT, DA), dtype=jnp.float32)

    mesh = plsc.VectorSubcoreMesh(core_axis_name="c", subcore_axis_name="s",
                                  num_cores=NC, num_subcores=NS)
    partials = pl.kernel(
        _seg_sum_kernel,
        out_type=jax.ShapeDtypeStruct((NC, N, DA), jnp.float32),
        mesh=mesh,
        scratch_types=[
            pltpu.VMEM((K,), jnp.int32),
            pltpu.VMEM((K,), jnp.int32),
            pltpu.VMEM((K, DA), jnp.float32),
            pltpu.VMEM_SHARED((N, DA), jnp.float32),
            pltpu.SemaphoreType.DMA,
        ],
        compiler_params=pltpu.CompilerParams(use_tc_tiling_on_sc=False),
    )(taug, src, dst, zeros_h)

    out = pl.pallas_call(
        _finish_kernel,
        grid=(N // rows_blk,),
        in_specs=[
            pl.BlockSpec((NC, rows_blk, DA), lambda i: (0, i, 0)),
            pl.BlockSpec((D, D), lambda i: (0, 0)),
            pl.BlockSpec(memory_space=pltpu.SMEM),
        ],
        out_specs=pl.BlockSpec((rows_blk, D), lambda i: (i, 0)),
        out_shape=jax.ShapeDtypeStruct((N, D), jnp.float32),
    )(partials, gc_weight, c2d)

    return out


# speedup vs baseline: 9.3645x; 1.6576x over previous
import functools

import jax
import jax.numpy as jnp
from jax import lax
from jax.experimental import pallas as pl
from jax.experimental.pallas import tpu as pltpu
from jax.experimental.pallas import tpu_sc as plsc

N = 10000
E = 320000
D = 128
DA = 144
NC = 2
NS = 16
NW = NC * NS
EPT = E // NW
K = 80
NCHUNK = EPT // K
RPT = N // NS


def _scale_kernel(emb_ref, c_ref, out_ref):
    y = emb_ref[...]
    c = c_ref[0, 0]
    rc = jnp.sqrt(c)
    n = jnp.sqrt(jnp.sum(y * y, axis=1, keepdims=True))
    n = jnp.clip(n, 1e-10)
    z = rc * n
    atanh = 0.5 * jnp.log((1.0 + z) / (1.0 - z))
    t = (2.0 / rc) * atanh / n * y
    rows = y.shape[0]
    ones = jnp.ones((rows, 1), dtype=y.dtype)
    pad = jnp.zeros((rows, DA - D - 1), dtype=y.dtype)
    out_ref[...] = jnp.concatenate([t, ones, pad], axis=1)


def _seg_sum_kernel(taug, src_h, dst_h, zeros_h, out,
                    dst_all, src_a, src_b, rows_a, rows_b,
                    acc_sh, sem_a, sem_b, semi_a, semi_b):
    cid = lax.axis_index("c")
    sid = lax.axis_index("s")
    tid = cid * NS + sid

    pltpu.sync_copy(zeros_h, acc_sh.at[pl.ds(sid * RPT, RPT)])
    pltpu.sync_copy(dst_h.at[tid], dst_all)
    plsc.subcore_barrier()

    pltpu.sync_copy(src_h.at[tid, 0], src_a)
    pltpu.sync_copy(src_h.at[tid, 1], src_b)
    pltpu.async_copy(taug.at[src_a], rows_a, sem_a)
    pltpu.async_copy(taug.at[src_b], rows_b, sem_b)

    def body(i2, carry):
        ia = 2 * i2
        ib = ia + 1
        pltpu.make_async_copy(taug.at[src_a], rows_a, sem_a).wait()
        pltpu.sync_copy(rows_a, acc_sh.at[dst_all.at[ia]], add=True)

        @pl.when(ia + 2 < NCHUNK)
        def _():
            pltpu.async_copy(src_h.at[tid, ia + 2], src_a, semi_a)

        pltpu.make_async_copy(taug.at[src_b], rows_b, sem_b).wait()
        pltpu.sync_copy(rows_b, acc_sh.at[dst_all.at[ib]], add=True)

        @pl.when(ib + 2 < NCHUNK)
        def _():
            pltpu.async_copy(src_h.at[tid, ib + 2], src_b, semi_b)

        @pl.when(ia + 2 < NCHUNK)
        def _():
            pltpu.make_async_copy(src_h.at[tid, ia + 2], src_a, semi_a).wait()
            pltpu.async_copy(taug.at[src_a], rows_a, sem_a)

        @pl.when(ib + 2 < NCHUNK)
        def _():
            pltpu.make_async_copy(src_h.at[tid, ib + 2], src_b, semi_b).wait()
            pltpu.async_copy(taug.at[src_b], rows_b, sem_b)

        return carry

    lax.fori_loop(0, NCHUNK // 2, body, 0)
    pltpu.make_async_copy(taug.at[src_a], rows_a, sem_a).wait()
    pltpu.sync_copy(rows_a, acc_sh.at[dst_all.at[NCHUNK - 1]], add=True)

    plsc.subcore_barrier()
    pltpu.sync_copy(acc_sh.at[pl.ds(sid * RPT, RPT)],
                    out.at[cid, pl.ds(sid * RPT, RPT)])


def _finish_kernel(part_ref, w_ref, c_ref, out_ref):
    p = part_ref[...]
    s = p[0] + p[1]
    c = c_ref[0, 0]
    rc = jnp.sqrt(c)
    agg = s[:, :D]
    cnt = jnp.clip(s[:, D:D + 1], 1.0)
    neigh = jnp.dot(agg, w_ref[...], preferred_element_type=jnp.float32) / cnt
    m = jnp.sqrt(jnp.sum(neigh * neigh, axis=1, keepdims=True))
    m = jnp.clip(m, 1e-10)
    out_ref[...] = jnp.tanh(rc * m * 0.5) * neigh / (rc * m)


def kernel(edge_index, node_embeddings, gc_weight, curvature):
    c2d = curvature.reshape(1, 1).astype(jnp.float32)
    src = edge_index[0].astype(jnp.int32).reshape(NW, NCHUNK, K)
    dst = edge_index[1].astype(jnp.int32).reshape(NW, NCHUNK, K)

    rows_blk = 1000
    taug = pl.pallas_call(
        _scale_kernel,
        grid=(N // rows_blk,),
        in_specs=[
            pl.BlockSpec((rows_blk, D), lambda i: (i, 0)),
            pl.BlockSpec(memory_space=pltpu.SMEM),
        ],
        out_specs=pl.BlockSpec((rows_blk, DA), lambda i: (i, 0)),
        out_shape=jax.ShapeDtypeStruct((N, DA), jnp.float32),
    )(node_embeddings, c2d)

    zeros_h = jnp.zeros((RPT, DA), dtype=jnp.float32)

    mesh = plsc.VectorSubcoreMesh(core_axis_name="c", subcore_axis_name="s",
                                  num_cores=NC, num_subcores=NS)
    partials = pl.kernel(
        _seg_sum_kernel,
        out_type=jax.ShapeDtypeStruct((NC, N, DA), jnp.float32),
        mesh=mesh,
        scratch_types=[
            pltpu.VMEM((NCHUNK, K), jnp.int32),
            pltpu.VMEM((K,), jnp.int32),
            pltpu.VMEM((K,), jnp.int32),
            pltpu.VMEM((K, DA), jnp.float32),
            pltpu.VMEM((K, DA), jnp.float32),
            pltpu.VMEM_SHARED((N, DA), jnp.float32),
            pltpu.SemaphoreType.DMA,
            pltpu.SemaphoreType.DMA,
            pltpu.SemaphoreType.DMA,
            pltpu.SemaphoreType.DMA,
        ],
        compiler_params=pltpu.CompilerParams(use_tc_tiling_on_sc=False),
    )(taug, src, dst, zeros_h)

    out = pl.pallas_call(
        _finish_kernel,
        grid=(N // rows_blk,),
        in_specs=[
            pl.BlockSpec((NC, rows_blk, DA), lambda i: (0, i, 0)),
            pl.BlockSpec((D, D), lambda i: (0, 0)),
            pl.BlockSpec(memory_space=pltpu.SMEM),
        ],
        out_specs=pl.BlockSpec((rows_blk, D), lambda i: (i, 0)),
        out_shape=jax.ShapeDtypeStruct((N, D), jnp.float32),
    )(partials, gc_weight, c2d)

    return out
